# baseline (speedup 1.0000x reference)
import functools

import jax
import jax.numpy as jnp
from jax import lax
from jax.experimental import pallas as pl
from jax.experimental.pallas import tpu as pltpu
from jax.experimental.pallas import tpu_sc as plsc

N = 8192
NW = 32
CPW = N // NW
FK = 64
NFRAG = N * 64
IDX_LEN = 2 * N + FK


def _bf16_round(v):
    t = plsc.bitcast(v, jnp.int32)
    c = jnp.bitwise_and(lax.shift_right_logical(t, 16), 1)
    r = jnp.bitwise_and(t + c + 0x7FFF, jnp.int32(-65536))
    return plsc.bitcast(r, jnp.float32)


def _gather_sum_body(f_hbm, v_hbm, out_hbm, f_v, idx_v, acc, rowbuf, sem0, sem1):
    nc = 2
    wid = lax.axis_index("s") * nc + lax.axis_index("c")

    pltpu.sync_copy(f_hbm, f_v)

    def _zi(i, _):
        idx_v[pl.ds(pl.multiple_of(i * 16, 16), 16)] = jnp.zeros((16,), jnp.int32)
        return 0
    lax.fori_loop(0, IDX_LEN // 16, _zi, 0)

    for s in range(CPW // 16):
        acc[pl.ds(s * 16, 16)] = jnp.zeros((16,), jnp.float32)

    def _cg(g, cnt):
        v = f_v[pl.ds(pl.multiple_of(g * 16, 16), 16)]
        m = v > 0.5
        inc = m.astype(jnp.int32)
        p = plsc.cumsum(inc) - inc + cnt
        j = lax.iota(jnp.int32, 16) + g * 16
        e0 = (lax.shift_right_logical(j, 3) * 512
              + jnp.bitwise_and(j, 7) + 16 * wid)
        plsc.store_scatter(idx_v, [2 * p], e0, mask=m)
        plsc.store_scatter(idx_v, [2 * p + 1], e0 + 8, mask=m)
        return cnt + jnp.sum(inc)
    cnt = lax.fori_loop(0, N // 16, _cg, jnp.int32(0))

    ne = 2 * cnt
    nfull = ne // FK
    nch = (ne + (FK - 1)) // FK

    def _issue(c, b, sem):
        off = pl.multiple_of(c * FK, FK)
        pltpu.make_async_copy(
            v_hbm.at[idx_v.at[pl.ds(off, FK)]], rowbuf.at[b], sem).start()

    def _wait(b, sem):
        pltpu.make_async_copy(
            v_hbm.at[idx_v.at[pl.ds(0, FK)]], rowbuf.at[b], sem).wait()

    def _accumulate(c, b):
        @pl.when(c < nfull)
        def _():
            def _acc_body(s, _):
                o = pl.multiple_of(s * 32, 32)
                a00 = acc[pl.ds(o, 16)]
                a01 = acc[pl.ds(o + 16, 16)]
                a10 = acc[pl.ds(o + 128, 16)]
                a11 = acc[pl.ds(o + 144, 16)]
                for rp in range(FK // 2):
                    a00 = a00 + _bf16_round(rowbuf[b, 2 * rp, pl.ds(o, 16)])
                    a01 = a01 + _bf16_round(rowbuf[b, 2 * rp, pl.ds(o + 16, 16)])
                    a10 = a10 + _bf16_round(rowbuf[b, 2 * rp + 1, pl.ds(o, 16)])
                    a11 = a11 + _bf16_round(rowbuf[b, 2 * rp + 1, pl.ds(o + 16, 16)])
                acc[pl.ds(o, 16)] = a00
                acc[pl.ds(o + 16, 16)] = a01
                acc[pl.ds(o + 128, 16)] = a10
                acc[pl.ds(o + 144, 16)] = a11
                return 0
            lax.fori_loop(0, 4, _acc_body, 0)

        @pl.when(c >= nfull)
        def _():
            zero = jnp.zeros((16,), jnp.float32)

            def _acc_body(s, _):
                o = pl.multiple_of(s * 32, 32)
                a00 = acc[pl.ds(o, 16)]
                a01 = acc[pl.ds(o + 16, 16)]
                a10 = acc[pl.ds(o + 128, 16)]
                a11 = acc[pl.ds(o + 144, 16)]
                for rp in range(FK // 2):
                    v0 = jnp.full((16,), c * FK + 2 * rp < ne)
                    v1 = jnp.full((16,), c * FK + 2 * rp + 1 < ne)
                    a00 = a00 + jnp.where(
                        v0, _bf16_round(rowbuf[b, 2 * rp, pl.ds(o, 16)]), zero)
                    a01 = a01 + jnp.where(
                        v0, _bf16_round(rowbuf[b, 2 * rp, pl.ds(o + 16, 16)]), zero)
                    a10 = a10 + jnp.where(
                        v1, _bf16_round(rowbuf[b, 2 * rp + 1, pl.ds(o, 16)]), zero)
                    a11 = a11 + jnp.where(
                        v1, _bf16_round(rowbuf[b, 2 * rp + 1, pl.ds(o + 16, 16)]), zero)
                acc[pl.ds(o, 16)] = a00
                acc[pl.ds(o + 16, 16)] = a01
                acc[pl.ds(o + 128, 16)] = a10
                acc[pl.ds(o + 144, 16)] = a11
                return 0
            lax.fori_loop(0, 4, _acc_body, 0)

    @pl.when(nch > 0)
    def _():
        _issue(0, 0, sem0)

    @pl.when(nch > 1)
    def _():
        _issue(1, 1, sem1)

    def _body2(c2, _):
        c0 = 2 * c2
        c1 = c0 + 1

        @pl.when(c0 < nch)
        def _():
            _wait(0, sem0)
            _accumulate(c0, 0)

        @pl.when(c0 + 2 < nch)
        def _():
            _issue(c0 + 2, 0, sem0)

        @pl.when(c1 < nch)
        def _():
            _wait(1, sem1)
            _accumulate(c1, 1)

        @pl.when(c1 + 2 < nch)
        def _():
            _issue(c1 + 2, 1, sem1)

        return 0

    lax.fori_loop(0, (nch + 1) // 2, _body2, 0)

    pltpu.sync_copy(acc, out_hbm.at[pl.ds(pl.multiple_of(wid * CPW, CPW), CPW)])


def _sc_node_x(is_firing, V):
    mesh = plsc.VectorSubcoreMesh(core_axis_name="c", subcore_axis_name="s")
    k = functools.partial(
        pl.kernel,
        mesh=mesh,
        out_type=jax.ShapeDtypeStruct((N,), jnp.float32),
        scratch_types=[
            pltpu.VMEM((N,), jnp.float32),
            pltpu.VMEM((IDX_LEN,), jnp.int32),
            pltpu.VMEM((CPW,), jnp.float32),
            pltpu.VMEM((2, FK, 128), jnp.float32),
            pltpu.SemaphoreType.DMA,
            pltpu.SemaphoreType.DMA,
        ],
        compiler_params=pltpu.CompilerParams(needs_layout_passes=False),
    )(_gather_sum_body)
    return k(is_firing, V)


def _threshold_body(x_ref, nx_ref, nf_ref):
    v = x_ref[...]
    theta = jnp.mean(v)
    nx_ref[...] = v
    nf_ref[...] = (v > theta).astype(jnp.float32)


def kernel(x, is_firing, W):
    V = (W.reshape(N // 8, 8, 64, 128)
          .transpose(0, 2, 1, 3)
          .reshape(NFRAG, 128))
    node_x = _sc_node_x(is_firing, V)
    nx, nf = pl.pallas_call(
        _threshold_body,
        out_shape=(
            jax.ShapeDtypeStruct((8, N // 8), jnp.float32),
            jax.ShapeDtypeStruct((8, N // 8), jnp.float32),
        ),
    )(node_x.reshape(8, N // 8))
    return nx.reshape(N), nf.reshape(N)


# speedup vs baseline: 7.7534x; 7.7534x over previous
import jax
import jax.numpy as jnp
from jax.experimental import pallas as pl
from jax.experimental.pallas import tpu as pltpu

N = 8192
BLK = 512
STEPS = N // BLK


def _fused_body(f_ref, w_ref, nx_ref, nf_ref, stage):
    i = pl.program_id(0)
    partial = jax.lax.dot_general(
        w_ref[...], f_ref[...],
        dimension_numbers=(((1,), (0,)), ((), ())),
        preferred_element_type=jnp.float32,
    )
    rows = BLK // 128
    stage[pl.ds(i * rows, rows), :] = partial.reshape(rows, 128)

    @pl.when(i == STEPS - 1)
    def _():
        v = stage[...]
        theta = jnp.mean(v)
        nx_ref[...] = v
        nf_ref[...] = (v > theta).astype(jnp.float32)


def kernel(x, is_firing, W):
    f2 = is_firing.reshape(N, 1)
    nx, nf = pl.pallas_call(
        _fused_body,
        grid=(STEPS,),
        in_specs=[
            pl.BlockSpec((N, 1), lambda i: (0, 0)),
            pl.BlockSpec((BLK, N), lambda i: (i, 0)),
        ],
        out_specs=(
            pl.BlockSpec((N // 128, 128), lambda i: (0, 0)),
            pl.BlockSpec((N // 128, 128), lambda i: (0, 0)),
        ),
        out_shape=(
            jax.ShapeDtypeStruct((N // 128, 128), jnp.float32),
            jax.ShapeDtypeStruct((N // 128, 128), jnp.float32),
        ),
        scratch_shapes=[pltpu.VMEM((N // 128, 128), jnp.float32)],
    )(f2, W)
    return nx.reshape(N), nf.reshape(N)


# speedup vs baseline: 7.7772x; 1.0031x over previous
import jax
import jax.numpy as jnp
from jax.experimental import pallas as pl
from jax.experimental.pallas import tpu as pltpu

N = 8192
BLK = 256
STEPS = N // BLK


def _fused_body(f_ref, w_ref, nx_ref, nf_ref, stage):
    i = pl.program_id(0)
    partial = jax.lax.dot_general(
        w_ref[...], f_ref[...],
        dimension_numbers=(((1,), (0,)), ((), ())),
        preferred_element_type=jnp.float32,
    )
    rows = BLK // 128
    stage[pl.ds(i * rows, rows), :] = partial.reshape(rows, 128)

    @pl.when(i == STEPS - 1)
    def _():
        v = stage[...]
        theta = jnp.mean(v)
        nx_ref[...] = v
        nf_ref[...] = (v > theta).astype(jnp.float32)


def kernel(x, is_firing, W):
    f2 = is_firing.reshape(N, 1)
    nx, nf = pl.pallas_call(
        _fused_body,
        grid=(STEPS,),
        in_specs=[
            pl.BlockSpec((N, 1), lambda i: (0, 0)),
            pl.BlockSpec((BLK, N), lambda i: (i, 0)),
        ],
        out_specs=(
            pl.BlockSpec((N // 128, 128), lambda i: (0, 0)),
            pl.BlockSpec((N // 128, 128), lambda i: (0, 0)),
        ),
        out_shape=(
            jax.ShapeDtypeStruct((N // 128, 128), jnp.float32),
            jax.ShapeDtypeStruct((N // 128, 128), jnp.float32),
        ),
        scratch_shapes=[pltpu.VMEM((N // 128, 128), jnp.float32)],

    )(f2, W)
    return nx.reshape(N), nf.reshape(N)
